# baseline (speedup 1.0000x reference)
import functools

import jax
import jax.numpy as jnp
from jax.experimental import pallas as pl
from jax.experimental.pallas import tpu as pltpu
from jax.experimental.pallas import tpu_sc as plsc

B, C, N, K = 32, 3, 1024, 32
CO = 64
CP = 128
CPAD = 8
BN_EPS = 1e-5
SENT = -1e37

_BF = jnp.bfloat16


def _dot32(a, b, dn):
    ah = a.astype(_BF)
    al = (a - ah.astype(jnp.float32)).astype(_BF)
    bh = b.astype(_BF)
    bl = (b - bh.astype(jnp.float32)).astype(_BF)
    f = lambda u, v: jax.lax.dot_general(u, v, dn, preferred_element_type=jnp.float32)
    return f(ah, bh) + f(ah, bl) + f(al, bh)


def _dot32_rhs(a_exact, b, dn):
    ah = a_exact.astype(_BF)
    bh = b.astype(_BF)
    bl = (b - bh.astype(jnp.float32)).astype(_BF)
    f = lambda u, v: jax.lax.dot_general(u, v, dn, preferred_element_type=jnp.float32)
    return f(ah, bh) + f(ah, bl)


def _p1_body(xp_ref, a8_ref, bma8_ref, idx_ref, q_ref, p_ref, st_ref, d_ref):
    b = pl.program_id(0)
    x2 = xp_ref[0]
    x2t = jnp.transpose(x2)
    xb = x2.astype(_BF).astype(jnp.float32)
    xbt = x2t.astype(_BF).astype(jnp.float32)
    g0 = (xb[:, 0:1] * xbt[0:1, :] + xb[:, 1:2] * xbt[1:2, :]
          + xb[:, 2:3] * xbt[2:3, :])
    xxc = jnp.sum(x2 * x2, axis=1, keepdims=True)
    xxr = jnp.sum(x2t * x2t, axis=0, keepdims=True)
    d_ref[...] = 2.0 * g0 - xxc - xxr

    a8 = a8_ref[...]
    bma8 = bma8_ref[...]
    q = (x2[:, 0:1] * a8[0:1, :] + x2[:, 1:2] * a8[1:2, :]
         + x2[:, 2:3] * a8[2:3, :])
    p = (x2[:, 0:1] * bma8[0:1, :] + x2[:, 1:2] * bma8[1:2, :]
         + x2[:, 2:3] * bma8[2:3, :])
    q_ref[0] = q
    p_ref[0] = p

    iota_m = jax.lax.broadcasted_iota(jnp.int32, (N, N), 0)

    def step(k, _):
        d = d_ref[...]
        mx = jnp.max(d, axis=0)
        cand = jnp.where(d == mx[None, :], iota_m, N)
        widx = jnp.min(cand, axis=0).astype(jnp.int32)
        idx_ref[0, pl.ds(k, 1), :] = (widx + b * N).reshape(1, N)
        d_ref[...] = jnp.where(iota_m == widx[None, :], SENT, d)
        return 0

    jax.lax.fori_loop(0, K, step, 0)

    m = (d_ref[...] <= SENT * 0.5).astype(jnp.float32)
    dn0 = (((0,), (0,)), ((), ()))
    mq = _dot32_rhs(m, q, dn0)
    mq2 = _dot32_rhs(m, q * q, dn0)
    kf = jnp.float32(K)
    s1 = jnp.sum(mq, axis=0) + kf * jnp.sum(p, axis=0)
    s2 = (jnp.sum(mq2, axis=0) + 2.0 * jnp.sum(p * mq, axis=0)
          + kf * jnp.sum(p * p, axis=0))
    st = jnp.stack([s1, s2], axis=0)

    @pl.when(b == 0)
    def _():
        st_ref[...] = st

    @pl.when(b != 0)
    def _():
        st_ref[...] += st


def _phase1(xp, a8, bma8):
    return pl.pallas_call(
        _p1_body,
        grid=(B,),
        in_specs=[
            pl.BlockSpec((1, N, CPAD), lambda b: (b, 0, 0)),
            pl.BlockSpec((CPAD, CP), lambda b: (0, 0)),
            pl.BlockSpec((CPAD, CP), lambda b: (0, 0)),
        ],
        out_specs=[
            pl.BlockSpec((1, K, N), lambda b: (b, 0, 0)),
            pl.BlockSpec((1, N, CP), lambda b: (b, 0, 0)),
            pl.BlockSpec((1, N, CP), lambda b: (b, 0, 0)),
            pl.BlockSpec((2, CP), lambda b: (0, 0)),
        ],
        out_shape=[
            jax.ShapeDtypeStruct((B, K, N), jnp.int32),
            jax.ShapeDtypeStruct((B, N, CP), jnp.float32),
            jax.ShapeDtypeStruct((B, N, CP), jnp.float32),
            jax.ShapeDtypeStruct((2, CP), jnp.float32),
        ],
        scratch_shapes=[pltpu.VMEM((N, N), jnp.float32)],
    )(xp, a8, bma8)


NIDX = B * K * N
_GW = 128


def _sc_gather(table, idxflat):
    mesh = plsc.VectorSubcoreMesh(core_axis_name="core", subcore_axis_name="subcore")

    @functools.partial(
        pl.kernel,
        out_type=jax.ShapeDtypeStruct((NIDX, CP), jnp.float32),
        mesh=mesh,
    )
    def gk(x_hbm, i_hbm, o_hbm):
        def body(i_vmem, o_vmem):
            pltpu.sync_copy(x_hbm.at[i_vmem.at[0]], o_vmem)

        pltpu.emit_pipeline(
            body,
            grid=(NIDX // _GW,),
            in_specs=[pl.BlockSpec((1, _GW), index_map=lambda i: (0, i))],
            out_specs=[pl.BlockSpec((_GW, CP), index_map=lambda i: (i, 0))],
            core_axis_name=("core", "subcore"),
            dimension_semantics=(pltpu.PARALLEL,),
        )(i_hbm, o_hbm)

    return gk(table, idxflat)


def _bn_affine(st_ref, gamma_ref, beta_ref):
    cnt = jnp.float32(B * N * K)
    mean = st_ref[0, :] / cnt
    var = st_ref[1, :] / cnt - mean * mean
    s = gamma_ref[0] * jax.lax.rsqrt(var + BN_EPS)
    t = beta_ref[0] - mean * s
    return s, t


def _leaky(y):
    return jnp.where(y >= 0, y, 0.2 * y)


BN3 = 256


def _p3_body(g_ref, p_ref, st1_ref, g1_ref, b1_ref, szz_ref, sz_ref):
    i = pl.program_id(0)
    j = pl.program_id(1)
    s1, t1 = _bn_affine(st1_ref, g1_ref, b1_ref)
    z = _leaky((g_ref[0] + p_ref[0]) * s1[None, None, :] + t1[None, None, :])
    z2 = z.reshape(K * BN3, CP)
    ztz = _dot32(z2, z2, (((0,), (0,)), ((), ())))
    zs = jnp.sum(z2, axis=0).reshape(1, CP)

    @pl.when((i == 0) & (j == 0))
    def _():
        szz_ref[...] = ztz
        sz_ref[...] = zs

    @pl.when((i != 0) | (j != 0))
    def _():
        szz_ref[...] += ztz
        sz_ref[...] += zs


def _phase3(g4, p4, st1, g1r, b1r):
    return pl.pallas_call(
        _p3_body,
        grid=(B, N // BN3),
        in_specs=[
            pl.BlockSpec((1, K, BN3, CP), lambda i, j: (i, 0, j, 0)),
            pl.BlockSpec((1, 1, BN3, CP), lambda i, j: (i, 0, j, 0)),
            pl.BlockSpec((2, CP), lambda i, j: (0, 0)),
            pl.BlockSpec((1, CP), lambda i, j: (0, 0)),
            pl.BlockSpec((1, CP), lambda i, j: (0, 0)),
        ],
        out_specs=[
            pl.BlockSpec((CP, CP), lambda i, j: (0, 0)),
            pl.BlockSpec((1, CP), lambda i, j: (0, 0)),
        ],
        out_shape=[
            jax.ShapeDtypeStruct((CP, CP), jnp.float32),
            jax.ShapeDtypeStruct((1, CP), jnp.float32),
        ],
    )(g4, p4, st1, g1r, b1r)


def _p4_body(g_ref, p_ref, st1_ref, g1_ref, b1_ref, szz_ref, sz_ref,
             g2_ref, b2_ref, w2p_ref, w2tp_ref, o_ref):
    cnt = jnp.float32(B * N * K)
    dnm = (((1,), (0,)), ((), ()))
    s1, t1 = _bn_affine(st1_ref, g1_ref, b1_ref)
    mean2 = _dot32(sz_ref[...], w2tp_ref[...], dnm)[0] / cnt
    w2p = w2p_ref[...]
    ey2sq = jnp.sum(_dot32(w2p, szz_ref[...], dnm) * w2p, axis=1) / cnt
    var2 = ey2sq - mean2 * mean2
    s2 = g2_ref[0] * jax.lax.rsqrt(var2 + BN_EPS)
    t2 = b2_ref[0] - mean2 * s2
    z = _leaky((g_ref[0] + p_ref[0]) * s1[None, None, :] + t1[None, None, :])
    y2 = _dot32(z.reshape(K * BN3, CP), w2tp_ref[...], dnm)
    o = _leaky(y2.reshape(K, BN3, CO) * s2[None, None, :] + t2[None, None, :])
    o_ref[0] = jnp.max(o, axis=0)


def _phase4(g4, p4, st1, g1r, b1r, szz, sz, g2r, b2r, w2p, w2tp):
    return pl.pallas_call(
        _p4_body,
        grid=(B, N // BN3),
        in_specs=[
            pl.BlockSpec((1, K, BN3, CP), lambda i, j: (i, 0, j, 0)),
            pl.BlockSpec((1, 1, BN3, CP), lambda i, j: (i, 0, j, 0)),
            pl.BlockSpec((2, CP), lambda i, j: (0, 0)),
            pl.BlockSpec((1, CP), lambda i, j: (0, 0)),
            pl.BlockSpec((1, CP), lambda i, j: (0, 0)),
            pl.BlockSpec((CP, CP), lambda i, j: (0, 0)),
            pl.BlockSpec((1, CP), lambda i, j: (0, 0)),
            pl.BlockSpec((1, CO), lambda i, j: (0, 0)),
            pl.BlockSpec((1, CO), lambda i, j: (0, 0)),
            pl.BlockSpec((CO, CP), lambda i, j: (0, 0)),
            pl.BlockSpec((CP, CO), lambda i, j: (0, 0)),
        ],
        out_specs=pl.BlockSpec((1, BN3, CO), lambda i, j: (i, j, 0)),
        out_shape=jax.ShapeDtypeStruct((B, N, CO), jnp.float32),
    )(g4, p4, st1, g1r, b1r, szz, sz, g2r, b2r, w2p, w2tp)


def kernel(x, W1, g1, b1, W2, g2, b2):
    xt = jnp.transpose(x, (0, 2, 1))
    xp = jnp.pad(xt, ((0, 0), (0, 0), (0, CPAD - C)))
    a = W1[:, :C]
    bma = W1[:, C:] - a
    a8 = jnp.pad(a.T, ((0, CPAD - C), (0, CP - CO)))
    bma8 = jnp.pad(bma.T, ((0, CPAD - C), (0, CP - CO)))
    g1r = jnp.pad(g1.reshape(1, CO), ((0, 0), (0, CP - CO)))
    b1r = jnp.pad(b1.reshape(1, CO), ((0, 0), (0, CP - CO)))
    g2r, b2r = g2.reshape(1, CO), b2.reshape(1, CO)
    w2p = jnp.pad(W2, ((0, 0), (0, CP - CO)))
    w2tp = jnp.pad(W2.T, ((0, CP - CO), (0, 0)))

    idx, q, p, st1 = _phase1(xp, a8, bma8)

    g = _sc_gather(q.reshape(B * N, CP), idx.reshape(1, NIDX))
    g4 = g.reshape(B, K, N, CP)
    p4 = p.reshape(B, 1, N, CP)

    szz, sz = _phase3(g4, p4, st1, g1r, b1r)
    out = _phase4(g4, p4, st1, g1r, b1r, szz, sz, g2r, b2r, w2p, w2tp)
    return jnp.transpose(out, (0, 2, 1))


# speedup vs baseline: 1.2133x; 1.2133x over previous
import functools

import jax
import jax.numpy as jnp
from jax.experimental import pallas as pl
from jax.experimental.pallas import tpu as pltpu
from jax.experimental.pallas import tpu_sc as plsc

B, C, N, K = 32, 3, 1024, 32
CO = 64
CP = 128
CPAD = 8
BN_EPS = 1e-5
SENT = -1e37

_BF = jnp.bfloat16


def _dot32(a, b, dn):
    ah = a.astype(_BF)
    al = (a - ah.astype(jnp.float32)).astype(_BF)
    bh = b.astype(_BF)
    bl = (b - bh.astype(jnp.float32)).astype(_BF)
    f = lambda u, v: jax.lax.dot_general(u, v, dn, preferred_element_type=jnp.float32)
    return f(ah, bh) + f(ah, bl) + f(al, bh)


def _dot32_rhs(a_exact, b, dn):
    ah = a_exact.astype(_BF)
    bh = b.astype(_BF)
    bl = (b - bh.astype(jnp.float32)).astype(_BF)
    f = lambda u, v: jax.lax.dot_general(u, v, dn, preferred_element_type=jnp.float32)
    return f(ah, bh) + f(ah, bl)


def _p1_body(xp_ref, a8_ref, bma8_ref, idx_ref, q_ref, p_ref, st_ref, d_ref,
             i_ref):
    b = pl.program_id(0)
    x2 = xp_ref[0]
    x2t = jnp.transpose(x2)
    xb = x2.astype(_BF).astype(jnp.float32)
    xbt = x2t.astype(_BF).astype(jnp.float32)
    g0 = (xb[:, 0:1] * xbt[0:1, :] + xb[:, 1:2] * xbt[1:2, :]
          + xb[:, 2:3] * xbt[2:3, :])
    xxc = jnp.sum(x2 * x2, axis=1, keepdims=True)
    xxr = jnp.sum(x2t * x2t, axis=0, keepdims=True)
    d_ref[...] = 2.0 * g0 - xxc - xxr

    a8 = a8_ref[...]
    bma8 = bma8_ref[...]
    q = (x2[:, 0:1] * a8[0:1, :] + x2[:, 1:2] * a8[1:2, :]
         + x2[:, 2:3] * a8[2:3, :])
    p = (x2[:, 0:1] * bma8[0:1, :] + x2[:, 1:2] * bma8[1:2, :]
         + x2[:, 2:3] * bma8[2:3, :])
    q_ref[0] = q
    p_ref[0] = p

    i_ref[...] = jax.lax.broadcasted_iota(jnp.int32, (N, N), 0)

    def step(k, prev):
        dm = jnp.where(i_ref[...] == prev[None, :], SENT, d_ref[...])
        d_ref[...] = dm
        widx = jnp.argmax(dm, axis=0).astype(jnp.int32)
        idx_ref[0, pl.ds(k, 1), :] = (widx + b * N).reshape(1, N)
        return widx

    prev0 = jnp.full((N,), N, jnp.int32)
    last = jax.lax.fori_loop(0, K, step, prev0)
    d_ref[...] = jnp.where(i_ref[...] == last[None, :], SENT, d_ref[...])

    m = (d_ref[...] <= SENT * 0.5).astype(jnp.float32)
    dn0 = (((0,), (0,)), ((), ()))
    mq = _dot32_rhs(m, q, dn0)
    mq2 = _dot32_rhs(m, q * q, dn0)
    kf = jnp.float32(K)
    s1 = jnp.sum(mq, axis=0) + kf * jnp.sum(p, axis=0)
    s2 = (jnp.sum(mq2, axis=0) + 2.0 * jnp.sum(p * mq, axis=0)
          + kf * jnp.sum(p * p, axis=0))
    st = jnp.stack([s1, s2], axis=0)

    @pl.when(b == 0)
    def _():
        st_ref[...] = st

    @pl.when(b != 0)
    def _():
        st_ref[...] += st


def _phase1(xp, a8, bma8):
    return pl.pallas_call(
        _p1_body,
        grid=(B,),
        in_specs=[
            pl.BlockSpec((1, N, CPAD), lambda b: (b, 0, 0)),
            pl.BlockSpec((CPAD, CP), lambda b: (0, 0)),
            pl.BlockSpec((CPAD, CP), lambda b: (0, 0)),
        ],
        out_specs=[
            pl.BlockSpec((1, K, N), lambda b: (b, 0, 0)),
            pl.BlockSpec((1, N, CP), lambda b: (b, 0, 0)),
            pl.BlockSpec((1, N, CP), lambda b: (b, 0, 0)),
            pl.BlockSpec((2, CP), lambda b: (0, 0)),
        ],
        out_shape=[
            jax.ShapeDtypeStruct((B, K, N), jnp.int32),
            jax.ShapeDtypeStruct((B, N, CP), jnp.float32),
            jax.ShapeDtypeStruct((B, N, CP), jnp.float32),
            jax.ShapeDtypeStruct((2, CP), jnp.float32),
        ],
        scratch_shapes=[pltpu.VMEM((N, N), jnp.float32),
                        pltpu.VMEM((N, N), jnp.int32)],
    )(xp, a8, bma8)


NIDX = B * K * N
_GW = 128


def _sc_gather(table, idxflat):
    mesh = plsc.VectorSubcoreMesh(core_axis_name="core", subcore_axis_name="subcore")

    @functools.partial(
        pl.kernel,
        out_type=jax.ShapeDtypeStruct((NIDX, CP), jnp.float32),
        mesh=mesh,
    )
    def gk(x_hbm, i_hbm, o_hbm):
        def body(i_vmem, o_vmem):
            pltpu.sync_copy(x_hbm.at[i_vmem.at[0]], o_vmem)

        pltpu.emit_pipeline(
            body,
            grid=(NIDX // _GW,),
            in_specs=[pl.BlockSpec((1, _GW), index_map=lambda i: (0, i))],
            out_specs=[pl.BlockSpec((_GW, CP), index_map=lambda i: (i, 0))],
            core_axis_name=("core", "subcore"),
            dimension_semantics=(pltpu.PARALLEL,),
        )(i_hbm, o_hbm)

    return gk(table, idxflat)


def _bn_affine(st_ref, gamma_ref, beta_ref):
    cnt = jnp.float32(B * N * K)
    mean = st_ref[0, :] / cnt
    var = st_ref[1, :] / cnt - mean * mean
    s = gamma_ref[0] * jax.lax.rsqrt(var + BN_EPS)
    t = beta_ref[0] - mean * s
    return s, t


def _leaky(y):
    return jnp.where(y >= 0, y, 0.2 * y)


BN3 = 256


def _p3_body(g_ref, p_ref, st1_ref, g1_ref, b1_ref, szz_ref, sz_ref):
    i = pl.program_id(0)
    j = pl.program_id(1)
    s1, t1 = _bn_affine(st1_ref, g1_ref, b1_ref)
    z = _leaky((g_ref[0] + p_ref[0]) * s1[None, None, :] + t1[None, None, :])
    z2 = z.reshape(K * BN3, CP)
    ztz = _dot32(z2, z2, (((0,), (0,)), ((), ())))
    zs = jnp.sum(z2, axis=0).reshape(1, CP)

    @pl.when((i == 0) & (j == 0))
    def _():
        szz_ref[...] = ztz
        sz_ref[...] = zs

    @pl.when((i != 0) | (j != 0))
    def _():
        szz_ref[...] += ztz
        sz_ref[...] += zs


def _phase3(g4, p4, st1, g1r, b1r):
    return pl.pallas_call(
        _p3_body,
        grid=(B, N // BN3),
        in_specs=[
            pl.BlockSpec((1, K, BN3, CP), lambda i, j: (i, 0, j, 0)),
            pl.BlockSpec((1, 1, BN3, CP), lambda i, j: (i, 0, j, 0)),
            pl.BlockSpec((2, CP), lambda i, j: (0, 0)),
            pl.BlockSpec((1, CP), lambda i, j: (0, 0)),
            pl.BlockSpec((1, CP), lambda i, j: (0, 0)),
        ],
        out_specs=[
            pl.BlockSpec((CP, CP), lambda i, j: (0, 0)),
            pl.BlockSpec((1, CP), lambda i, j: (0, 0)),
        ],
        out_shape=[
            jax.ShapeDtypeStruct((CP, CP), jnp.float32),
            jax.ShapeDtypeStruct((1, CP), jnp.float32),
        ],
    )(g4, p4, st1, g1r, b1r)


def _p4_body(g_ref, p_ref, st1_ref, g1_ref, b1_ref, szz_ref, sz_ref,
             g2_ref, b2_ref, w2p_ref, w2tp_ref, o_ref):
    cnt = jnp.float32(B * N * K)
    dnm = (((1,), (0,)), ((), ()))
    s1, t1 = _bn_affine(st1_ref, g1_ref, b1_ref)
    mean2 = _dot32(sz_ref[...], w2tp_ref[...], dnm)[0] / cnt
    w2p = w2p_ref[...]
    ey2sq = jnp.sum(_dot32(w2p, szz_ref[...], dnm) * w2p, axis=1) / cnt
    var2 = ey2sq - mean2 * mean2
    s2 = g2_ref[0] * jax.lax.rsqrt(var2 + BN_EPS)
    t2 = b2_ref[0] - mean2 * s2
    z = _leaky((g_ref[0] + p_ref[0]) * s1[None, None, :] + t1[None, None, :])
    y2 = _dot32(z.reshape(K * BN3, CP), w2tp_ref[...], dnm)
    o = _leaky(y2.reshape(K, BN3, CO) * s2[None, None, :] + t2[None, None, :])
    o_ref[0] = jnp.max(o, axis=0)


def _phase4(g4, p4, st1, g1r, b1r, szz, sz, g2r, b2r, w2p, w2tp):
    return pl.pallas_call(
        _p4_body,
        grid=(B, N // BN3),
        in_specs=[
            pl.BlockSpec((1, K, BN3, CP), lambda i, j: (i, 0, j, 0)),
            pl.BlockSpec((1, 1, BN3, CP), lambda i, j: (i, 0, j, 0)),
            pl.BlockSpec((2, CP), lambda i, j: (0, 0)),
            pl.BlockSpec((1, CP), lambda i, j: (0, 0)),
            pl.BlockSpec((1, CP), lambda i, j: (0, 0)),
            pl.BlockSpec((CP, CP), lambda i, j: (0, 0)),
            pl.BlockSpec((1, CP), lambda i, j: (0, 0)),
            pl.BlockSpec((1, CO), lambda i, j: (0, 0)),
            pl.BlockSpec((1, CO), lambda i, j: (0, 0)),
            pl.BlockSpec((CO, CP), lambda i, j: (0, 0)),
            pl.BlockSpec((CP, CO), lambda i, j: (0, 0)),
        ],
        out_specs=pl.BlockSpec((1, BN3, CO), lambda i, j: (i, j, 0)),
        out_shape=jax.ShapeDtypeStruct((B, N, CO), jnp.float32),
    )(g4, p4, st1, g1r, b1r, szz, sz, g2r, b2r, w2p, w2tp)


def kernel(x, W1, g1, b1, W2, g2, b2):
    xt = jnp.transpose(x, (0, 2, 1))
    xp = jnp.pad(xt, ((0, 0), (0, 0), (0, CPAD - C)))
    a = W1[:, :C]
    bma = W1[:, C:] - a
    a8 = jnp.pad(a.T, ((0, CPAD - C), (0, CP - CO)))
    bma8 = jnp.pad(bma.T, ((0, CPAD - C), (0, CP - CO)))
    g1r = jnp.pad(g1.reshape(1, CO), ((0, 0), (0, CP - CO)))
    b1r = jnp.pad(b1.reshape(1, CO), ((0, 0), (0, CP - CO)))
    g2r, b2r = g2.reshape(1, CO), b2.reshape(1, CO)
    w2p = jnp.pad(W2, ((0, 0), (0, CP - CO)))
    w2tp = jnp.pad(W2.T, ((0, CP - CO), (0, 0)))

    idx, q, p, st1 = _phase1(xp, a8, bma8)

    g = _sc_gather(q.reshape(B * N, CP), idx.reshape(1, NIDX))
    g4 = g.reshape(B, K, N, CP)
    p4 = p.reshape(B, 1, N, CP)

    szz, sz = _phase3(g4, p4, st1, g1r, b1r)
    out = _phase4(g4, p4, st1, g1r, b1r, szz, sz, g2r, b2r, w2p, w2tp)
    return jnp.transpose(out, (0, 2, 1))


# speedup vs baseline: 1.2537x; 1.0333x over previous
import functools

import jax
import jax.numpy as jnp
from jax.experimental import pallas as pl
from jax.experimental.pallas import tpu as pltpu
from jax.experimental.pallas import tpu_sc as plsc

B, C, N, K = 32, 3, 1024, 32
CO = 64
CP = 128
CPAD = 8
BN_EPS = 1e-5
SENT = -1e37

_BF = jnp.bfloat16


def _dot32(a, b, dn):
    ah = a.astype(_BF)
    al = (a - ah.astype(jnp.float32)).astype(_BF)
    bh = b.astype(_BF)
    bl = (b - bh.astype(jnp.float32)).astype(_BF)
    f = lambda u, v: jax.lax.dot_general(u, v, dn, preferred_element_type=jnp.float32)
    return f(ah, bh) + f(ah, bl) + f(al, bh)


def _dot32_rhs(a_exact, b, dn):
    ah = a_exact.astype(_BF)
    bh = b.astype(_BF)
    bl = (b - bh.astype(jnp.float32)).astype(_BF)
    f = lambda u, v: jax.lax.dot_general(u, v, dn, preferred_element_type=jnp.float32)
    return f(ah, bh) + f(ah, bl)


def _p1_body(xp_ref, a8_ref, bma8_ref, idx_ref, q_ref, p_ref, st_ref, d_ref,
             i_ref):
    b = pl.program_id(0)
    x2 = xp_ref[0]
    x2t = jnp.transpose(x2)
    xb = x2.astype(_BF).astype(jnp.float32)
    xbt = x2t.astype(_BF).astype(jnp.float32)
    g0 = (xb[:, 0:1] * xbt[0:1, :] + xb[:, 1:2] * xbt[1:2, :]
          + xb[:, 2:3] * xbt[2:3, :])
    xxc = jnp.sum(x2 * x2, axis=1, keepdims=True)
    xxr = jnp.sum(x2t * x2t, axis=0, keepdims=True)
    d_ref[...] = 2.0 * g0 - xxc - xxr

    a8 = a8_ref[...]
    bma8 = bma8_ref[...]
    q = (x2[:, 0:1] * a8[0:1, :] + x2[:, 1:2] * a8[1:2, :]
         + x2[:, 2:3] * a8[2:3, :])
    p = (x2[:, 0:1] * bma8[0:1, :] + x2[:, 1:2] * bma8[1:2, :]
         + x2[:, 2:3] * bma8[2:3, :])
    q_ref[0] = q
    p_ref[0] = p
    q = q[:, :CO]

    i_ref[...] = jax.lax.broadcasted_iota(jnp.int32, (N, N), 0)

    def step(k, prev):
        dm = jnp.where(i_ref[...] == prev[None, :], SENT, d_ref[...])
        d_ref[...] = dm
        widx = jnp.argmax(dm, axis=0).astype(jnp.int32)
        idx_ref[0, pl.ds(k, 1), :] = (widx + b * N).reshape(1, N)
        return widx

    prev0 = jnp.full((N,), N, jnp.int32)
    last = jax.lax.fori_loop(0, K, step, prev0)
    d_ref[...] = jnp.where(i_ref[...] == last[None, :], SENT, d_ref[...])

    m = (d_ref[...] <= SENT * 0.5).astype(jnp.float32)
    dn0 = (((0,), (0,)), ((), ()))
    mq = _dot32_rhs(m, q, dn0)
    mq2 = _dot32_rhs(m, q * q, dn0)
    kf = jnp.float32(K)
    s1 = jnp.sum(mq, axis=0) + kf * jnp.sum(p, axis=0)
    s2 = (jnp.sum(mq2, axis=0) + 2.0 * jnp.sum(p * mq, axis=0)
          + kf * jnp.sum(p * p, axis=0))
    st = jnp.stack([s1, s2], axis=0)

    @pl.when(b == 0)
    def _():
        st_ref[...] = st

    @pl.when(b != 0)
    def _():
        st_ref[...] += st


def _phase1(xp, a8, bma8):
    return pl.pallas_call(
        _p1_body,
        grid=(B,),
        in_specs=[
            pl.BlockSpec((1, N, CPAD), lambda b: (b, 0, 0)),
            pl.BlockSpec((CPAD, CP), lambda b: (0, 0)),
            pl.BlockSpec((CPAD, CO), lambda b: (0, 0)),
        ],
        out_specs=[
            pl.BlockSpec((1, K, N), lambda b: (b, 0, 0)),
            pl.BlockSpec((1, N, CP), lambda b: (b, 0, 0)),
            pl.BlockSpec((1, N, CO), lambda b: (b, 0, 0)),
            pl.BlockSpec((2, CO), lambda b: (0, 0)),
        ],
        out_shape=[
            jax.ShapeDtypeStruct((B, K, N), jnp.int32),
            jax.ShapeDtypeStruct((B, N, CP), jnp.float32),
            jax.ShapeDtypeStruct((B, N, CO), jnp.float32),
            jax.ShapeDtypeStruct((2, CO), jnp.float32),
        ],
        scratch_shapes=[pltpu.VMEM((N, N), jnp.float32),
                        pltpu.VMEM((N, N), jnp.int32)],
    )(xp, a8, bma8)


NIDX = B * K * N
_GW = 128


def _sc_gather(table, idxflat):
    mesh = plsc.VectorSubcoreMesh(core_axis_name="core", subcore_axis_name="subcore")

    @functools.partial(
        pl.kernel,
        out_type=jax.ShapeDtypeStruct((NIDX, CP), jnp.float32),
        mesh=mesh,
    )
    def gk(x_hbm, i_hbm, o_hbm):
        def body(i_vmem, o_vmem):
            pltpu.sync_copy(x_hbm.at[i_vmem.at[0]], o_vmem)

        pltpu.emit_pipeline(
            body,
            grid=(NIDX // _GW,),
            in_specs=[pl.BlockSpec((1, _GW), index_map=lambda i: (0, i))],
            out_specs=[pl.BlockSpec((_GW, CP), index_map=lambda i: (i, 0))],
            core_axis_name=("core", "subcore"),
            dimension_semantics=(pltpu.PARALLEL,),
        )(i_hbm, o_hbm)

    return gk(table, idxflat)


def _bn_affine(st_ref, gamma_ref, beta_ref):
    cnt = jnp.float32(B * N * K)
    mean = st_ref[0, :] / cnt
    var = st_ref[1, :] / cnt - mean * mean
    s = gamma_ref[0] * jax.lax.rsqrt(var + BN_EPS)
    t = beta_ref[0] - mean * s
    return s, t


def _leaky(y):
    return jnp.where(y >= 0, y, 0.2 * y)


BN3 = 256


def _p3_body(g_ref, p_ref, st1_ref, g1_ref, b1_ref, szz_ref, sz_ref):
    i = pl.program_id(0)
    j = pl.program_id(1)
    s1, t1 = _bn_affine(st1_ref, g1_ref, b1_ref)
    z = _leaky((g_ref[0][..., :CO] + p_ref[0]) * s1[None, None, :]
               + t1[None, None, :])
    z2 = z.reshape(K * BN3, CO)
    ztz = _dot32(z2, z2, (((0,), (0,)), ((), ())))
    zs = jnp.sum(z2, axis=0).reshape(1, CO)

    @pl.when((i == 0) & (j == 0))
    def _():
        szz_ref[...] = ztz
        sz_ref[...] = zs

    @pl.when((i != 0) | (j != 0))
    def _():
        szz_ref[...] += ztz
        sz_ref[...] += zs


def _phase3(g4, p4, st1, g1r, b1r):
    return pl.pallas_call(
        _p3_body,
        grid=(B, N // BN3),
        in_specs=[
            pl.BlockSpec((1, K, BN3, CP), lambda i, j: (i, 0, j, 0)),
            pl.BlockSpec((1, 1, BN3, CO), lambda i, j: (i, 0, j, 0)),
            pl.BlockSpec((2, CO), lambda i, j: (0, 0)),
            pl.BlockSpec((1, CO), lambda i, j: (0, 0)),
            pl.BlockSpec((1, CO), lambda i, j: (0, 0)),
        ],
        out_specs=[
            pl.BlockSpec((CO, CO), lambda i, j: (0, 0)),
            pl.BlockSpec((1, CO), lambda i, j: (0, 0)),
        ],
        out_shape=[
            jax.ShapeDtypeStruct((CO, CO), jnp.float32),
            jax.ShapeDtypeStruct((1, CO), jnp.float32),
        ],
    )(g4, p4, st1, g1r, b1r)


def _p4_body(g_ref, p_ref, st1_ref, g1_ref, b1_ref, szz_ref, sz_ref,
             g2_ref, b2_ref, w2p_ref, w2tp_ref, o_ref):
    cnt = jnp.float32(B * N * K)
    dnm = (((1,), (0,)), ((), ()))
    s1, t1 = _bn_affine(st1_ref, g1_ref, b1_ref)
    mean2 = _dot32(sz_ref[...], w2tp_ref[...], dnm)[0] / cnt
    w2p = w2p_ref[...]
    ey2sq = jnp.sum(_dot32(w2p, szz_ref[...], dnm) * w2p, axis=1) / cnt
    var2 = ey2sq - mean2 * mean2
    s2 = g2_ref[0] * jax.lax.rsqrt(var2 + BN_EPS)
    t2 = b2_ref[0] - mean2 * s2
    z = _leaky((g_ref[0][..., :CO] + p_ref[0]) * s1[None, None, :]
               + t1[None, None, :])
    y2 = _dot32(z.reshape(K * BN3, CO), w2tp_ref[...], dnm)
    o = _leaky(y2.reshape(K, BN3, CO) * s2[None, None, :] + t2[None, None, :])
    o_ref[0] = jnp.max(o, axis=0)


def _phase4(g4, p4, st1, g1r, b1r, szz, sz, g2r, b2r, w2p, w2tp):
    return pl.pallas_call(
        _p4_body,
        grid=(B, N // BN3),
        in_specs=[
            pl.BlockSpec((1, K, BN3, CP), lambda i, j: (i, 0, j, 0)),
            pl.BlockSpec((1, 1, BN3, CO), lambda i, j: (i, 0, j, 0)),
            pl.BlockSpec((2, CO), lambda i, j: (0, 0)),
            pl.BlockSpec((1, CO), lambda i, j: (0, 0)),
            pl.BlockSpec((1, CO), lambda i, j: (0, 0)),
            pl.BlockSpec((CO, CO), lambda i, j: (0, 0)),
            pl.BlockSpec((1, CO), lambda i, j: (0, 0)),
            pl.BlockSpec((1, CO), lambda i, j: (0, 0)),
            pl.BlockSpec((1, CO), lambda i, j: (0, 0)),
            pl.BlockSpec((CO, CO), lambda i, j: (0, 0)),
            pl.BlockSpec((CO, CO), lambda i, j: (0, 0)),
        ],
        out_specs=pl.BlockSpec((1, BN3, CO), lambda i, j: (i, j, 0)),
        out_shape=jax.ShapeDtypeStruct((B, N, CO), jnp.float32),
    )(g4, p4, st1, g1r, b1r, szz, sz, g2r, b2r, w2p, w2tp)


def kernel(x, W1, g1, b1, W2, g2, b2):
    xt = jnp.transpose(x, (0, 2, 1))
    xp = jnp.pad(xt, ((0, 0), (0, 0), (0, CPAD - C)))
    a = W1[:, :C]
    bma = W1[:, C:] - a
    a8 = jnp.pad(a.T, ((0, CPAD - C), (0, CP - CO)))
    bma8 = jnp.pad(bma.T, ((0, CPAD - C), (0, 0)))
    g1r, b1r = g1.reshape(1, CO), b1.reshape(1, CO)
    g2r, b2r = g2.reshape(1, CO), b2.reshape(1, CO)
    w2p = W2
    w2tp = W2.T

    idx, q, p, st1 = _phase1(xp, a8, bma8)

    g = _sc_gather(q.reshape(B * N, CP), idx.reshape(1, NIDX))
    g4 = g.reshape(B, K, N, CP)
    p4 = p.reshape(B, 1, N, CO)

    szz, sz = _phase3(g4, p4, st1, g1r, b1r)
    out = _phase4(g4, p4, st1, g1r, b1r, szz, sz, g2r, b2r, w2p, w2tp)
    return jnp.transpose(out, (0, 2, 1))


# speedup vs baseline: 1.5425x; 1.2303x over previous
import functools

import jax
import jax.numpy as jnp
from jax.experimental import pallas as pl
from jax.experimental.pallas import tpu as pltpu
from jax.experimental.pallas import tpu_sc as plsc

B, C, N, K = 32, 3, 1024, 32
CO = 64
CP = 128
CPAD = 8
BN_EPS = 1e-5
SENT = -1e37

_BF = jnp.bfloat16


def _dot32(a, b, dn):
    ah = a.astype(_BF)
    al = (a - ah.astype(jnp.float32)).astype(_BF)
    bh = b.astype(_BF)
    bl = (b - bh.astype(jnp.float32)).astype(_BF)
    f = lambda u, v: jax.lax.dot_general(u, v, dn, preferred_element_type=jnp.float32)
    return f(ah, bh) + f(ah, bl) + f(al, bh)


def _dot32_rhs(a_exact, b, dn):
    ah = a_exact.astype(_BF)
    bh = b.astype(_BF)
    bl = (b - bh.astype(jnp.float32)).astype(_BF)
    f = lambda u, v: jax.lax.dot_general(u, v, dn, preferred_element_type=jnp.float32)
    return f(ah, bh) + f(ah, bl)


def _p1_body(xp_ref, a8_ref, bma8_ref, idx_ref, q_ref, p_ref, st_ref, d_ref,
             i_ref):
    b = pl.program_id(0)
    x2 = xp_ref[0]
    x2t = jnp.transpose(x2)
    xb = x2.astype(_BF).astype(jnp.float32)
    xbt = x2t.astype(_BF).astype(jnp.float32)
    g0 = (xb[:, 0:1] * xbt[0:1, :] + xb[:, 1:2] * xbt[1:2, :]
          + xb[:, 2:3] * xbt[2:3, :])
    xxc = jnp.sum(x2 * x2, axis=1, keepdims=True)
    xxr = jnp.sum(x2t * x2t, axis=0, keepdims=True)
    d_ref[...] = 2.0 * g0 - xxc - xxr

    a8 = a8_ref[...]
    bma8 = bma8_ref[...]
    q = (x2[:, 0:1] * a8[0:1, :] + x2[:, 1:2] * a8[1:2, :]
         + x2[:, 2:3] * a8[2:3, :])
    p = (x2[:, 0:1] * bma8[0:1, :] + x2[:, 1:2] * bma8[1:2, :]
         + x2[:, 2:3] * bma8[2:3, :])
    q_ref[0] = q
    p_ref[0] = p
    q = q[:, :CO]

    i_ref[...] = jax.lax.broadcasted_iota(jnp.int32, (N, N), 0)

    def step(k, prev):
        dm = jnp.where(i_ref[...] == prev[None, :], SENT, d_ref[...])
        d_ref[...] = dm
        widx = jnp.argmax(dm, axis=0).astype(jnp.int32)
        idx_ref[0, pl.ds(k, 1), :] = (widx + b * N).reshape(1, N)
        return widx

    prev0 = jnp.full((N,), N, jnp.int32)
    last = jax.lax.fori_loop(0, K, step, prev0)
    d_ref[...] = jnp.where(i_ref[...] == last[None, :], SENT, d_ref[...])

    m = (d_ref[...] <= SENT * 0.5).astype(jnp.float32)
    dn0 = (((0,), (0,)), ((), ()))
    mq = _dot32_rhs(m, q, dn0)
    mq2 = _dot32_rhs(m, q * q, dn0)
    kf = jnp.float32(K)
    s1 = jnp.sum(mq, axis=0) + kf * jnp.sum(p, axis=0)
    s2 = (jnp.sum(mq2, axis=0) + 2.0 * jnp.sum(p * mq, axis=0)
          + kf * jnp.sum(p * p, axis=0))
    st = jnp.stack([s1, s2], axis=0)

    @pl.when(b == 0)
    def _():
        st_ref[...] = st

    @pl.when(b != 0)
    def _():
        st_ref[...] += st


NCH = 4
BC = B // NCH


def _phase1(xp, a8, bma8):
    return pl.pallas_call(
        _p1_body,
        grid=(BC,),
        in_specs=[
            pl.BlockSpec((1, N, CPAD), lambda b: (b, 0, 0)),
            pl.BlockSpec((CPAD, CP), lambda b: (0, 0)),
            pl.BlockSpec((CPAD, CO), lambda b: (0, 0)),
        ],
        out_specs=[
            pl.BlockSpec((1, K, N), lambda b: (b, 0, 0)),
            pl.BlockSpec((1, N, CP), lambda b: (b, 0, 0)),
            pl.BlockSpec((1, N, CO), lambda b: (b, 0, 0)),
            pl.BlockSpec((2, CO), lambda b: (0, 0)),
        ],
        out_shape=[
            jax.ShapeDtypeStruct((BC, K, N), jnp.int32),
            jax.ShapeDtypeStruct((BC, N, CP), jnp.float32),
            jax.ShapeDtypeStruct((BC, N, CO), jnp.float32),
            jax.ShapeDtypeStruct((2, CO), jnp.float32),
        ],
        scratch_shapes=[pltpu.VMEM((N, N), jnp.float32),
                        pltpu.VMEM((N, N), jnp.int32)],
    )(xp, a8, bma8)


NIDXC = BC * K * N
_GW = 128


def _sc_gather(table, idxflat):
    mesh = plsc.VectorSubcoreMesh(core_axis_name="core", subcore_axis_name="subcore")

    @functools.partial(
        pl.kernel,
        out_type=jax.ShapeDtypeStruct((NIDXC, CP), jnp.float32),
        mesh=mesh,
    )
    def gk(x_hbm, i_hbm, o_hbm):
        def body(i_vmem, o_vmem):
            pltpu.sync_copy(x_hbm.at[i_vmem.at[0]], o_vmem)

        pltpu.emit_pipeline(
            body,
            grid=(NIDXC // _GW,),
            in_specs=[pl.BlockSpec((1, _GW), index_map=lambda i: (0, i))],
            out_specs=[pl.BlockSpec((_GW, CP), index_map=lambda i: (i, 0))],
            core_axis_name=("core", "subcore"),
            dimension_semantics=(pltpu.PARALLEL,),
        )(i_hbm, o_hbm)

    return gk(table, idxflat)


def _bn_affine(st_ref, gamma_ref, beta_ref):
    cnt = jnp.float32(B * N * K)
    mean = st_ref[0, :] / cnt
    var = st_ref[1, :] / cnt - mean * mean
    s = gamma_ref[0] * jax.lax.rsqrt(var + BN_EPS)
    t = beta_ref[0] - mean * s
    return s, t


def _leaky(y):
    return jnp.where(y >= 0, y, 0.2 * y)


BN3 = 256


def _p3_body(g_ref, p_ref, st1_ref, g1_ref, b1_ref, szz_ref, sz_ref):
    i = pl.program_id(0)
    j = pl.program_id(1)
    s1, t1 = _bn_affine(st1_ref, g1_ref, b1_ref)
    z = _leaky((g_ref[0][..., :CO] + p_ref[0]) * s1[None, None, :]
               + t1[None, None, :])
    z2 = z.reshape(K * BN3, CO)
    ztz = _dot32(z2, z2, (((0,), (0,)), ((), ())))
    zs = jnp.sum(z2, axis=0).reshape(1, CO)

    @pl.when((i == 0) & (j == 0))
    def _():
        szz_ref[...] = ztz
        sz_ref[...] = zs

    @pl.when((i != 0) | (j != 0))
    def _():
        szz_ref[...] += ztz
        sz_ref[...] += zs


def _phase3(g4, p4, st1, g1r, b1r):
    return pl.pallas_call(
        _p3_body,
        grid=(BC, N // BN3),
        in_specs=[
            pl.BlockSpec((1, K, BN3, CP), lambda i, j: (i, 0, j, 0)),
            pl.BlockSpec((1, 1, BN3, CO), lambda i, j: (i, 0, j, 0)),
            pl.BlockSpec((2, CO), lambda i, j: (0, 0)),
            pl.BlockSpec((1, CO), lambda i, j: (0, 0)),
            pl.BlockSpec((1, CO), lambda i, j: (0, 0)),
        ],
        out_specs=[
            pl.BlockSpec((CO, CO), lambda i, j: (0, 0)),
            pl.BlockSpec((1, CO), lambda i, j: (0, 0)),
        ],
        out_shape=[
            jax.ShapeDtypeStruct((CO, CO), jnp.float32),
            jax.ShapeDtypeStruct((1, CO), jnp.float32),
        ],
    )(g4, p4, st1, g1r, b1r)


def _p4_body(g_ref, p_ref, st1_ref, g1_ref, b1_ref, szz_ref, sz_ref,
             g2_ref, b2_ref, w2p_ref, w2tp_ref, o_ref):
    cnt = jnp.float32(B * N * K)
    dnm = (((1,), (0,)), ((), ()))
    s1, t1 = _bn_affine(st1_ref, g1_ref, b1_ref)
    mean2 = _dot32(sz_ref[...], w2tp_ref[...], dnm)[0] / cnt
    w2p = w2p_ref[...]
    ey2sq = jnp.sum(_dot32(w2p, szz_ref[...], dnm) * w2p, axis=1) / cnt
    var2 = ey2sq - mean2 * mean2
    s2 = g2_ref[0] * jax.lax.rsqrt(var2 + BN_EPS)
    t2 = b2_ref[0] - mean2 * s2
    z = _leaky((g_ref[0][..., :CO] + p_ref[0]) * s1[None, None, :]
               + t1[None, None, :])
    y2 = _dot32(z.reshape(K * BN3, CO), w2tp_ref[...], dnm)
    o = _leaky(y2.reshape(K, BN3, CO) * s2[None, None, :] + t2[None, None, :])
    o_ref[0] = jnp.max(o, axis=0)


def _phase4(g4, p4, st1, g1r, b1r, szz, sz, g2r, b2r, w2p, w2tp):
    return pl.pallas_call(
        _p4_body,
        grid=(BC, N // BN3),
        in_specs=[
            pl.BlockSpec((1, K, BN3, CP), lambda i, j: (i, 0, j, 0)),
            pl.BlockSpec((1, 1, BN3, CO), lambda i, j: (i, 0, j, 0)),
            pl.BlockSpec((2, CO), lambda i, j: (0, 0)),
            pl.BlockSpec((1, CO), lambda i, j: (0, 0)),
            pl.BlockSpec((1, CO), lambda i, j: (0, 0)),
            pl.BlockSpec((CO, CO), lambda i, j: (0, 0)),
            pl.BlockSpec((1, CO), lambda i, j: (0, 0)),
            pl.BlockSpec((1, CO), lambda i, j: (0, 0)),
            pl.BlockSpec((1, CO), lambda i, j: (0, 0)),
            pl.BlockSpec((CO, CO), lambda i, j: (0, 0)),
            pl.BlockSpec((CO, CO), lambda i, j: (0, 0)),
        ],
        out_specs=pl.BlockSpec((1, BN3, CO), lambda i, j: (i, j, 0)),
        out_shape=jax.ShapeDtypeStruct((BC, N, CO), jnp.float32),
    )(g4, p4, st1, g1r, b1r, szz, sz, g2r, b2r, w2p, w2tp)


def kernel(x, W1, g1, b1, W2, g2, b2):
    xt = jnp.transpose(x, (0, 2, 1))
    xp = jnp.pad(xt, ((0, 0), (0, 0), (0, CPAD - C)))
    a = W1[:, :C]
    bma = W1[:, C:] - a
    a8 = jnp.pad(a.T, ((0, CPAD - C), (0, CP - CO)))
    bma8 = jnp.pad(bma.T, ((0, CPAD - C), (0, 0)))
    g1r, b1r = g1.reshape(1, CO), b1.reshape(1, CO)
    g2r, b2r = g2.reshape(1, CO), b2.reshape(1, CO)
    w2p = W2
    w2tp = W2.T

    gs, ps, sts = [], [], []
    for c in range(NCH):
        idx_c, q_c, p_c, st_c = _phase1(xp[c * BC:(c + 1) * BC], a8, bma8)
        g_c = _sc_gather(q_c.reshape(BC * N, CP), idx_c.reshape(1, NIDXC))
        gs.append(g_c.reshape(BC, K, N, CP))
        ps.append(p_c.reshape(BC, 1, N, CO))
        sts.append(st_c)
    st1 = sts[0] + sts[1] + sts[2] + sts[3]

    stats = [_phase3(gs[c], ps[c], st1, g1r, b1r) for c in range(NCH)]
    szz = sum(szt[0] for szt in stats)
    sz = sum(szt[1] for szt in stats)

    outs = [_phase4(gs[c], ps[c], st1, g1r, b1r, szz, sz, g2r, b2r, w2p, w2tp)
            for c in range(NCH)]
    out = jnp.concatenate(outs, axis=0)
    return jnp.transpose(out, (0, 2, 1))


# speedup vs baseline: 1.6570x; 1.0742x over previous
import functools

import jax
import jax.numpy as jnp
from jax.experimental import pallas as pl
from jax.experimental.pallas import tpu as pltpu
from jax.experimental.pallas import tpu_sc as plsc

B, C, N, K = 32, 3, 1024, 32
CO = 64
CP = 128
CPAD = 8
BN_EPS = 1e-5
SENT = -1e37

_BF = jnp.bfloat16


def _dot32(a, b, dn):
    ah = a.astype(_BF)
    al = (a - ah.astype(jnp.float32)).astype(_BF)
    bh = b.astype(_BF)
    bl = (b - bh.astype(jnp.float32)).astype(_BF)
    f = lambda u, v: jax.lax.dot_general(u, v, dn, preferred_element_type=jnp.float32)
    return f(ah, bh) + f(ah, bl) + f(al, bh)


def _dot32_rhs(a_exact, b, dn):
    ah = a_exact.astype(_BF)
    bh = b.astype(_BF)
    bl = (b - bh.astype(jnp.float32)).astype(_BF)
    f = lambda u, v: jax.lax.dot_general(u, v, dn, preferred_element_type=jnp.float32)
    return f(ah, bh) + f(ah, bl)


def _p1_body(xp_ref, a8_ref, bma8_ref, idx_ref, q_ref, p_ref, st_ref, d_ref,
             i_ref):
    b = pl.program_id(0)
    x2 = xp_ref[0]
    x2t = jnp.transpose(x2)
    xb = x2.astype(_BF).astype(jnp.float32)
    xbt = x2t.astype(_BF).astype(jnp.float32)
    g0 = (xb[:, 0:1] * xbt[0:1, :] + xb[:, 1:2] * xbt[1:2, :]
          + xb[:, 2:3] * xbt[2:3, :])
    xxc = jnp.sum(x2 * x2, axis=1, keepdims=True)
    xxr = jnp.sum(x2t * x2t, axis=0, keepdims=True)
    d_ref[...] = 2.0 * g0 - xxc - xxr

    a8 = a8_ref[...]
    bma8 = bma8_ref[...]
    q = (x2[:, 0:1] * a8[0:1, :] + x2[:, 1:2] * a8[1:2, :]
         + x2[:, 2:3] * a8[2:3, :])
    p = (x2[:, 0:1] * bma8[0:1, :] + x2[:, 1:2] * bma8[1:2, :]
         + x2[:, 2:3] * bma8[2:3, :])
    q_ref[0] = q
    p_ref[0] = p
    q = q[:, :CO]

    i_ref[...] = jax.lax.broadcasted_iota(jnp.int32, (N, N), 0)

    def step(k, prev):
        dm = jnp.where(i_ref[...] == prev[None, :], SENT, d_ref[...])
        d_ref[...] = dm
        widx = jnp.argmax(dm, axis=0).astype(jnp.int32)
        idx_ref[0, pl.ds(k, 1), :] = (widx + b * N).reshape(1, N)
        return widx

    prev0 = jnp.full((N,), N, jnp.int32)
    last = jax.lax.fori_loop(0, K, step, prev0)
    d_ref[...] = jnp.where(i_ref[...] == last[None, :], SENT, d_ref[...])

    m = (d_ref[...] <= SENT * 0.5).astype(jnp.float32)
    dn0 = (((0,), (0,)), ((), ()))
    mq = _dot32_rhs(m, q, dn0)
    mq2 = _dot32_rhs(m, q * q, dn0)
    kf = jnp.float32(K)
    s1 = jnp.sum(mq, axis=0) + kf * jnp.sum(p, axis=0)
    s2 = (jnp.sum(mq2, axis=0) + 2.0 * jnp.sum(p * mq, axis=0)
          + kf * jnp.sum(p * p, axis=0))
    st = jnp.stack([s1, s2], axis=0)

    @pl.when(b == 0)
    def _():
        st_ref[...] = st

    @pl.when(b != 0)
    def _():
        st_ref[...] += st


NCH = 4
BC = B // NCH


def _phase1(xp, a8, bma8):
    return pl.pallas_call(
        _p1_body,
        grid=(BC,),
        in_specs=[
            pl.BlockSpec((1, N, CPAD), lambda b: (b, 0, 0)),
            pl.BlockSpec((CPAD, CP), lambda b: (0, 0)),
            pl.BlockSpec((CPAD, CO), lambda b: (0, 0)),
        ],
        out_specs=[
            pl.BlockSpec((1, K, N), lambda b: (b, 0, 0)),
            pl.BlockSpec((1, N, CP), lambda b: (b, 0, 0)),
            pl.BlockSpec((1, N, CO), lambda b: (b, 0, 0)),
            pl.BlockSpec((2, CO), lambda b: (0, 0)),
        ],
        out_shape=[
            jax.ShapeDtypeStruct((BC, K, N), jnp.int32),
            jax.ShapeDtypeStruct((BC, N, CP), jnp.float32),
            jax.ShapeDtypeStruct((BC, N, CO), jnp.float32),
            jax.ShapeDtypeStruct((2, CO), jnp.float32),
        ],
        scratch_shapes=[pltpu.VMEM((N, N), jnp.float32),
                        pltpu.VMEM((N, N), jnp.int32)],
    )(xp, a8, bma8)


NIDXC = BC * K * N
_GW = 128


def _sc_gather(table, idxflat):
    mesh = plsc.VectorSubcoreMesh(core_axis_name="core", subcore_axis_name="subcore")

    @functools.partial(
        pl.kernel,
        out_type=jax.ShapeDtypeStruct((NIDXC, CP), jnp.float32),
        mesh=mesh,
    )
    def gk(x_hbm, i_hbm, o_hbm):
        def body(i_vmem, o_vmem):
            pltpu.sync_copy(x_hbm.at[i_vmem.at[0]], o_vmem)

        pltpu.emit_pipeline(
            body,
            grid=(NIDXC // _GW,),
            in_specs=[pl.BlockSpec((1, _GW), index_map=lambda i: (0, i))],
            out_specs=[pl.BlockSpec((_GW, CP), index_map=lambda i: (i, 0))],
            core_axis_name=("core", "subcore"),
            dimension_semantics=(pltpu.PARALLEL,),
        )(i_hbm, o_hbm)

    return gk(table, idxflat)


def _bn_affine(st_ref, gamma_ref, beta_ref):
    cnt = jnp.float32(B * N * K)
    mean = st_ref[0, :] / cnt
    var = st_ref[1, :] / cnt - mean * mean
    s = gamma_ref[0] * jax.lax.rsqrt(var + BN_EPS)
    t = beta_ref[0] - mean * s
    return s, t


def _leaky(y):
    return jnp.where(y >= 0, y, 0.2 * y)


BN3 = 256


def _p3_body(g_ref, p_ref, st1_ref, g1_ref, b1_ref, szz_ref, sz_ref):
    i = pl.program_id(0)
    j = pl.program_id(1)
    s1, t1 = _bn_affine(st1_ref, g1_ref, b1_ref)
    z = _leaky((g_ref[0][..., :CO] + p_ref[0]) * s1[None, None, :]
               + t1[None, None, :])
    z2 = z.reshape(K * BN3, CO)
    z2b = z2.astype(_BF)
    ztz = jax.lax.dot_general(z2b, z2b, (((0,), (0,)), ((), ())),
                              preferred_element_type=jnp.float32)
    zs = jnp.sum(z2, axis=0).reshape(1, CO)

    @pl.when((i == 0) & (j == 0))
    def _():
        szz_ref[...] = ztz
        sz_ref[...] = zs

    @pl.when((i != 0) | (j != 0))
    def _():
        szz_ref[...] += ztz
        sz_ref[...] += zs


def _phase3(g4, p4, st1, g1r, b1r):
    return pl.pallas_call(
        _p3_body,
        grid=(BC, N // BN3),
        in_specs=[
            pl.BlockSpec((1, K, BN3, CP), lambda i, j: (i, 0, j, 0)),
            pl.BlockSpec((1, 1, BN3, CO), lambda i, j: (i, 0, j, 0)),
            pl.BlockSpec((2, CO), lambda i, j: (0, 0)),
            pl.BlockSpec((1, CO), lambda i, j: (0, 0)),
            pl.BlockSpec((1, CO), lambda i, j: (0, 0)),
        ],
        out_specs=[
            pl.BlockSpec((CO, CO), lambda i, j: (0, 0)),
            pl.BlockSpec((1, CO), lambda i, j: (0, 0)),
        ],
        out_shape=[
            jax.ShapeDtypeStruct((CO, CO), jnp.float32),
            jax.ShapeDtypeStruct((1, CO), jnp.float32),
        ],
    )(g4, p4, st1, g1r, b1r)


def _p4_body(g_ref, p_ref, st1_ref, g1_ref, b1_ref, szz_ref, sz_ref,
             g2_ref, b2_ref, w2p_ref, w2tp_ref, o_ref):
    cnt = jnp.float32(B * N * K)
    dnm = (((1,), (0,)), ((), ()))
    s1, t1 = _bn_affine(st1_ref, g1_ref, b1_ref)
    mean2 = _dot32(sz_ref[...], w2tp_ref[...], dnm)[0] / cnt
    w2p = w2p_ref[...]
    ey2sq = jnp.sum(_dot32(w2p, szz_ref[...], dnm) * w2p, axis=1) / cnt
    var2 = ey2sq - mean2 * mean2
    s2 = g2_ref[0] * jax.lax.rsqrt(var2 + BN_EPS)
    t2 = b2_ref[0] - mean2 * s2
    z = _leaky((g_ref[0][..., :CO] + p_ref[0]) * s1[None, None, :]
               + t1[None, None, :])
    y2 = jax.lax.dot_general(z.reshape(K * BN3, CO).astype(_BF),
                             w2tp_ref[...].astype(_BF), dnm,
                             preferred_element_type=jnp.float32)
    o = _leaky(y2.reshape(K, BN3, CO) * s2[None, None, :] + t2[None, None, :])
    o_ref[0] = jnp.max(o, axis=0)


def _phase4(g4, p4, st1, g1r, b1r, szz, sz, g2r, b2r, w2p, w2tp):
    return pl.pallas_call(
        _p4_body,
        grid=(BC, N // BN3),
        in_specs=[
            pl.BlockSpec((1, K, BN3, CP), lambda i, j: (i, 0, j, 0)),
            pl.BlockSpec((1, 1, BN3, CO), lambda i, j: (i, 0, j, 0)),
            pl.BlockSpec((2, CO), lambda i, j: (0, 0)),
            pl.BlockSpec((1, CO), lambda i, j: (0, 0)),
            pl.BlockSpec((1, CO), lambda i, j: (0, 0)),
            pl.BlockSpec((CO, CO), lambda i, j: (0, 0)),
            pl.BlockSpec((1, CO), lambda i, j: (0, 0)),
            pl.BlockSpec((1, CO), lambda i, j: (0, 0)),
            pl.BlockSpec((1, CO), lambda i, j: (0, 0)),
            pl.BlockSpec((CO, CO), lambda i, j: (0, 0)),
            pl.BlockSpec((CO, CO), lambda i, j: (0, 0)),
        ],
        out_specs=pl.BlockSpec((1, BN3, CO), lambda i, j: (i, j, 0)),
        out_shape=jax.ShapeDtypeStruct((BC, N, CO), jnp.float32),
    )(g4, p4, st1, g1r, b1r, szz, sz, g2r, b2r, w2p, w2tp)


def kernel(x, W1, g1, b1, W2, g2, b2):
    xt = jnp.transpose(x, (0, 2, 1))
    xp = jnp.pad(xt, ((0, 0), (0, 0), (0, CPAD - C)))
    a = W1[:, :C]
    bma = W1[:, C:] - a
    a8 = jnp.pad(a.T, ((0, CPAD - C), (0, CP - CO)))
    bma8 = jnp.pad(bma.T, ((0, CPAD - C), (0, 0)))
    g1r, b1r = g1.reshape(1, CO), b1.reshape(1, CO)
    g2r, b2r = g2.reshape(1, CO), b2.reshape(1, CO)
    w2p = W2
    w2tp = W2.T

    gs, ps, sts = [], [], []
    for c in range(NCH):
        idx_c, q_c, p_c, st_c = _phase1(xp[c * BC:(c + 1) * BC], a8, bma8)
        g_c = _sc_gather(q_c.reshape(BC * N, CP), idx_c.reshape(1, NIDXC))
        gs.append(g_c.reshape(BC, K, N, CP))
        ps.append(p_c.reshape(BC, 1, N, CO))
        sts.append(st_c)
    st1 = sts[0] + sts[1] + sts[2] + sts[3]

    stats = [_phase3(gs[c], ps[c], st1, g1r, b1r) for c in range(NCH)]
    szz = sum(szt[0] for szt in stats)
    sz = sum(szt[1] for szt in stats)

    outs = [_phase4(gs[c], ps[c], st1, g1r, b1r, szz, sz, g2r, b2r, w2p, w2tp)
            for c in range(NCH)]
    out = jnp.concatenate(outs, axis=0)
    return jnp.transpose(out, (0, 2, 1))
